# baseline (speedup 1.0000x reference)
import jax
import jax.numpy as jnp
from jax import lax
from jax.experimental import pallas as pl
from jax.experimental.pallas import tpu as pltpu
from jax.experimental.pallas import tpu_sc as plsc

_V = 1000000
_B = 16384
_K = 20
_D = 16
_L = 16
_NC = 2
_NS = 16
_NW = _NC * _NS
_BPW = _B // _NW
_CB = 32
_NCHUNK = _BPW // _CB
_NIW = 128
_NJ = _CB * _K // _NIW
_NROW = 1 + _K
_SCHUNK = _NROW * _CB


def _sc_body(cw_hbm, pw_hbm, nw_hbm, in_hbm, out_hbm, sc_hbm,
             cidx_v, pidx_v, nidx_v, cg_v, pg_v, ng_v,
             crow_v, prow_v, nrow_v, scr_v, sem):
  c = lax.axis_index("c")
  s = lax.axis_index("s")
  wid = s * _NC + c
  base = wid * _BPW

  @pl.loop(0, _NCHUNK)
  def _chunk(ci):
    b0 = base + ci * _CB
    pltpu.sync_copy(cw_hbm.at[pl.ds(b0, _CB)], cidx_v)
    pltpu.sync_copy(pw_hbm.at[pl.ds(b0, _CB)], pidx_v)
    pltpu.sync_copy(nw_hbm.at[pl.ds(b0 * _K, _CB * _K)], nidx_v)
    for t in range(_CB // _L):
      cg_v[pl.ds(t * _L, _L)] = cidx_v[pl.ds(t * _L, _L)] >> 3
      pg_v[pl.ds(t * _L, _L)] = pidx_v[pl.ds(t * _L, _L)] >> 3
    for t in range(_CB * _K // _L):
      ng_v[pl.ds(t * _L, _L)] = nidx_v[pl.ds(t * _L, _L)] >> 3
    copies = [
        pltpu.async_copy(in_hbm.at[cg_v], crow_v, sem),
        pltpu.async_copy(out_hbm.at[pg_v], prow_v, sem),
    ]
    for j in range(_NJ):
      copies.append(pltpu.async_copy(
          out_hbm.at[ng_v.at[pl.ds(j * _NIW, _NIW)]],
          nrow_v.at[pl.ds(j * _NIW, _NIW)], sem))
    for cp in copies:
      cp.wait()

    @pl.loop(0, _CB // _L)
    def _group(g):
      row0 = g * _L
      lane = lax.iota(jnp.int32, 16)
      rowi = row0 + lane
      rowk = rowi * _K
      csub = (cidx_v[pl.ds(row0, _L)] & 7) * _D
      psub = (pidx_v[pl.ds(row0, _L)] & 7) * _D
      ccols = [plsc.load_gather(crow_v, [rowi, csub + d]) for d in range(_D)]

      acc = ccols[0] * plsc.load_gather(prow_v, [rowi, psub])
      for d in range(1, _D):
        acc = acc + ccols[d] * plsc.load_gather(prow_v, [rowi, psub + d])
      scr_v[pl.ds(row0, _L)] = acc

      for k in range(_K):
        ri = rowk + k
        nsub = (plsc.load_gather(nidx_v, [ri]) & 7) * _D
        acc = ccols[0] * plsc.load_gather(nrow_v, [ri, nsub])
        for d in range(1, _D):
          acc = acc + ccols[d] * plsc.load_gather(nrow_v, [ri, nsub + d])
        scr_v[pl.ds((1 + k) * _CB + row0, _L)] = -acc

    pltpu.sync_copy(scr_v, sc_hbm.at[pl.ds((wid * _NCHUNK + ci) * _SCHUNK,
                                           _SCHUNK)])


_sc_gather = pl.kernel(
    _sc_body,
    out_type=jax.ShapeDtypeStruct((_NROW * _B,), jnp.float32),
    mesh=plsc.VectorSubcoreMesh(core_axis_name="c", subcore_axis_name="s"),
    compiler_params=pltpu.CompilerParams(
        needs_layout_passes=False, use_tc_tiling_on_sc=True),
    scratch_types=[
        pltpu.VMEM((_CB,), jnp.int32),
        pltpu.VMEM((_CB,), jnp.int32),
        pltpu.VMEM((_CB * _K,), jnp.int32),
        pltpu.VMEM((_CB,), jnp.int32),
        pltpu.VMEM((_CB,), jnp.int32),
        pltpu.VMEM((_CB * _K,), jnp.int32),
        pltpu.VMEM((_CB, 128), jnp.float32),
        pltpu.VMEM((_CB, 128), jnp.float32),
        pltpu.VMEM((_CB * _K, 128), jnp.float32),
        pltpu.VMEM((_SCHUNK,), jnp.float32),
        pltpu.SemaphoreType.DMA,
    ],
)


_NT = _V // 128
_TAIL = _V - _NT * 128


def _cv_body(wti_hbm, wto_hbm, tli_hbm, tlo_hbm, ri_hbm, ro_hbm,
             tb_v, ob_v, tl_v, sem):
  c = lax.axis_index("c")
  s = lax.axis_index("s")
  wid = s * _NC + c
  lo = wid * _NT // _NW
  hi = (wid + 1) * _NT // _NW
  lane = lax.iota(jnp.int32, 16)

  @pl.loop(lo, hi)
  def _tile(t):
    for (wt, r) in ((wti_hbm, ri_hbm), (wto_hbm, ro_hbm)):
      pltpu.sync_copy(wt.at[pl.ds(0, 8), pl.ds(t * 128, 128)],
                      tb_v.at[pl.ds(0, 8)])
      pltpu.sync_copy(wt.at[pl.ds(8, 8), pl.ds(t * 128, 128)],
                      tb_v.at[pl.ds(8, 8)])
      for v0 in range(0, 128, _L):
        ridx = (v0 + lane) >> 3
        cidx = ((v0 + lane) & 7) * _D
        for d in range(_D):
          plsc.store_scatter(ob_v, [ridx, cidx + d], tb_v[d, pl.ds(v0, _L)])
      pltpu.sync_copy(ob_v, r.at[pl.ds(t * _D, _D)])

  @pl.when(wid == _NW - 1)
  def _tail_blk():
    for (tl, r) in ((tli_hbm, ri_hbm), (tlo_hbm, ro_hbm)):
      pltpu.sync_copy(tl, tl_v)
      for lv in range(_TAIL):
        plsc.store_scatter(
            ob_v,
            [jnp.full((16,), lv >> 3, dtype=jnp.int32),
             (lv & 7) * _D + lane],
            tl_v[lv, :])
      pltpu.sync_copy(ob_v.at[pl.ds(0, _TAIL // 8)],
                      r.at[pl.ds(_NT * _D, _TAIL // 8)])


_convert = pl.kernel(
    _cv_body,
    out_type=(jax.ShapeDtypeStruct((_V // 8, 128), jnp.float32),
              jax.ShapeDtypeStruct((_V // 8, 128), jnp.float32)),
    mesh=plsc.VectorSubcoreMesh(core_axis_name="c", subcore_axis_name="s"),
    compiler_params=pltpu.CompilerParams(
        needs_layout_passes=False, use_tc_tiling_on_sc=True),
    scratch_types=[
        pltpu.VMEM((_D, 128), jnp.float32),
        pltpu.VMEM((_D, 128), jnp.float32),
        pltpu.VMEM((_TAIL, _D), jnp.float32),
        pltpu.SemaphoreType.DMA,
    ],
)


def _to_rows(in_embed, out_embed):
  return _convert(in_embed.T, out_embed.T,
                  in_embed[_NT * 128:], out_embed[_NT * 128:])


def _loss_body(s_ref, o_ref):
  x = s_ref[...]
  o_ref[0, 0] = -jnp.sum(jax.nn.log_sigmoid(x)) / _B


_loss_call = pl.pallas_call(
    _loss_body,
    out_shape=jax.ShapeDtypeStruct((1, 1), jnp.float32),
    out_specs=pl.BlockSpec(memory_space=pltpu.SMEM),
)


def kernel(center_words, pos_context_words, neg_context_words, in_embed, out_embed):
  cw = center_words.astype(jnp.int32)
  pw = pos_context_words.astype(jnp.int32)
  nw = neg_context_words.astype(jnp.int32).reshape(_B * _K)
  rin, rout = _to_rows(in_embed, out_embed)
  scores = _sc_gather(cw, pw, nw, rin, rout)
  return _loss_call(scores.reshape(_NROW * _B // 128, 128)).reshape(())


# speedup vs baseline: 2.0199x; 2.0199x over previous
import jax
import jax.numpy as jnp
from jax import lax
from jax.experimental import pallas as pl
from jax.experimental.pallas import tpu as pltpu
from jax.experimental.pallas import tpu_sc as plsc

_V = 1000000
_B = 16384
_K = 20
_D = 16
_L = 16
_NC = 2
_NS = 16
_NW = _NC * _NS
_BPW = _B // _NW
_CB = 32
_NCHUNK = _BPW // _CB
_NIW = 128
_NJ = _CB * _K // _NIW
_NROW = 1 + _K
_SCHUNK = _NROW * _CB


def _sc_body(cw_hbm, pw_hbm, nw_hbm, in_hbm, out_hbm, sc_hbm,
             cidx_v, pidx_v, nidx_v, cg_v, pg_v, ng_v,
             crow_v, prow_v, nrow_v, scr_v, sem):
  c = lax.axis_index("c")
  s = lax.axis_index("s")
  wid = s * _NC + c
  base = wid * _BPW

  @pl.loop(0, _NCHUNK)
  def _chunk(ci):
    b0 = base + ci * _CB
    pltpu.sync_copy(cw_hbm.at[pl.ds(b0, _CB)], cidx_v)
    pltpu.sync_copy(pw_hbm.at[pl.ds(b0, _CB)], pidx_v)
    pltpu.sync_copy(nw_hbm.at[pl.ds(b0 * _K, _CB * _K)], nidx_v)
    for t in range(_CB // _L):
      cg_v[pl.ds(t * _L, _L)] = cidx_v[pl.ds(t * _L, _L)] >> 3
      pg_v[pl.ds(t * _L, _L)] = pidx_v[pl.ds(t * _L, _L)] >> 3
    for t in range(_CB * _K // _L):
      ng_v[pl.ds(t * _L, _L)] = nidx_v[pl.ds(t * _L, _L)] >> 3
    copies = [
        pltpu.async_copy(in_hbm.at[cg_v], crow_v, sem),
        pltpu.async_copy(out_hbm.at[pg_v], prow_v, sem),
    ]
    for j in range(_NJ):
      copies.append(pltpu.async_copy(
          out_hbm.at[ng_v.at[pl.ds(j * _NIW, _NIW)]],
          nrow_v.at[pl.ds(j * _NIW, _NIW)], sem))
    for cp in copies:
      cp.wait()

    @pl.loop(0, _CB // _L)
    def _group(g):
      row0 = g * _L
      lane = lax.iota(jnp.int32, 16)
      rowi = row0 + lane
      rowk = rowi * _K
      csub = (cidx_v[pl.ds(row0, _L)] & 7) * _D
      psub = (pidx_v[pl.ds(row0, _L)] & 7) * _D
      ccols = [plsc.load_gather(crow_v, [rowi, csub + d]) for d in range(_D)]

      acc = ccols[0] * plsc.load_gather(prow_v, [rowi, psub])
      for d in range(1, _D):
        acc = acc + ccols[d] * plsc.load_gather(prow_v, [rowi, psub + d])
      scr_v[pl.ds(row0, _L)] = acc

      for k in range(_K):
        ri = rowk + k
        nsub = (plsc.load_gather(nidx_v, [ri]) & 7) * _D
        acc = ccols[0] * plsc.load_gather(nrow_v, [ri, nsub])
        for d in range(1, _D):
          acc = acc + ccols[d] * plsc.load_gather(nrow_v, [ri, nsub + d])
        scr_v[pl.ds((1 + k) * _CB + row0, _L)] = -acc

    pltpu.sync_copy(scr_v, sc_hbm.at[pl.ds((wid * _NCHUNK + ci) * _SCHUNK,
                                           _SCHUNK)])


_sc_gather = pl.kernel(
    _sc_body,
    out_type=jax.ShapeDtypeStruct((_NROW * _B,), jnp.float32),
    mesh=plsc.VectorSubcoreMesh(core_axis_name="c", subcore_axis_name="s"),
    compiler_params=pltpu.CompilerParams(
        needs_layout_passes=False, use_tc_tiling_on_sc=True),
    scratch_types=[
        pltpu.VMEM((_CB,), jnp.int32),
        pltpu.VMEM((_CB,), jnp.int32),
        pltpu.VMEM((_CB * _K,), jnp.int32),
        pltpu.VMEM((_CB,), jnp.int32),
        pltpu.VMEM((_CB,), jnp.int32),
        pltpu.VMEM((_CB * _K,), jnp.int32),
        pltpu.VMEM((_CB, 128), jnp.float32),
        pltpu.VMEM((_CB, 128), jnp.float32),
        pltpu.VMEM((_CB * _K, 128), jnp.float32),
        pltpu.VMEM((_SCHUNK,), jnp.float32),
        pltpu.SemaphoreType.DMA,
    ],
)


_NT = _V // 128
_TAIL = _V - _NT * 128


_TPW = _NT // _NW
_NTB = 8


def _cv_group(tabs, tbs, obs, sem, t0, ntb, lane):
  copies = []
  for (wt, _), tb in zip(tabs, tbs):
    for dg in (0, 1):
      copies.append(pltpu.async_copy(
          wt.at[pl.ds(dg * 8, 8), pl.ds(t0 * 128, ntb * 128)],
          tb.at[pl.ds(dg * 8, 8), pl.ds(0, ntb * 128)], sem))
  for cp in copies:
    cp.wait()

  @pl.loop(0, ntb)
  def _tt(tt):
    for tb, ob in zip(tbs, obs):
      for v0 in range(0, 128, _L):
        ridx = tt * _D + ((v0 + lane) >> 3)
        cidx = ((v0 + lane) & 7) * _D
        for d in range(_D):
          plsc.store_scatter(ob, [ridx, cidx + d],
                             tb[d, pl.ds(tt * 128 + v0, _L)])

  for (_, r), ob in zip(tabs, obs):
    pltpu.sync_copy(ob.at[pl.ds(0, ntb * _D)], r.at[pl.ds(t0 * _D, ntb * _D)])


def _cv_body(wti_hbm, wto_hbm, tli_hbm, tlo_hbm, ri_hbm, ro_hbm,
             tbi_v, tbo_v, obi_v, obo_v, tl_v, sem):
  c = lax.axis_index("c")
  s = lax.axis_index("s")
  wid = s * _NC + c
  base = wid * _TPW
  lane = lax.iota(jnp.int32, 16)
  tabs = ((wti_hbm, ri_hbm), (wto_hbm, ro_hbm))
  tbs = (tbi_v, tbo_v)
  obs = (obi_v, obo_v)

  @pl.loop(0, _TPW // _NTB)
  def _g(g):
    _cv_group(tabs, tbs, obs, sem, base + g * _NTB, _NTB, lane)

  rem = _TPW % _NTB
  if rem:
    _cv_group(tabs, tbs, obs, sem, base + _TPW - rem, rem, lane)

  @pl.when(wid == _NW - 1)
  def _tail_blk():
    _cv_group(tabs, tbs, obs, sem, _NW * _TPW, _NT - _NW * _TPW, lane)
    for (tl, r), ob in zip(((tli_hbm, ri_hbm), (tlo_hbm, ro_hbm)), obs):
      pltpu.sync_copy(tl, tl_v)
      for lv in range(_TAIL):
        plsc.store_scatter(
            ob,
            [jnp.full((16,), lv >> 3, dtype=jnp.int32),
             (lv & 7) * _D + lane],
            tl_v[lv, :])
      pltpu.sync_copy(ob.at[pl.ds(0, _TAIL // 8)],
                      r.at[pl.ds(_NT * _D, _TAIL // 8)])


_convert = pl.kernel(
    _cv_body,
    out_type=(jax.ShapeDtypeStruct((_V // 8, 128), jnp.float32),
              jax.ShapeDtypeStruct((_V // 8, 128), jnp.float32)),
    mesh=plsc.VectorSubcoreMesh(core_axis_name="c", subcore_axis_name="s"),
    compiler_params=pltpu.CompilerParams(
        needs_layout_passes=False, use_tc_tiling_on_sc=True),
    scratch_types=[
        pltpu.VMEM((_D, _NTB * 128), jnp.float32),
        pltpu.VMEM((_D, _NTB * 128), jnp.float32),
        pltpu.VMEM((_NTB * _D, 128), jnp.float32),
        pltpu.VMEM((_NTB * _D, 128), jnp.float32),
        pltpu.VMEM((_TAIL, _D), jnp.float32),
        pltpu.SemaphoreType.DMA,
    ],
)


def _to_rows(in_embed, out_embed):
  return _convert(in_embed.T, out_embed.T,
                  in_embed[_NT * 128:], out_embed[_NT * 128:])


def _loss_body(s_ref, o_ref):
  x = s_ref[...]
  o_ref[0, 0] = -jnp.sum(jax.nn.log_sigmoid(x)) / _B


_loss_call = pl.pallas_call(
    _loss_body,
    out_shape=jax.ShapeDtypeStruct((1, 1), jnp.float32),
    out_specs=pl.BlockSpec(memory_space=pltpu.SMEM),
)


def kernel(center_words, pos_context_words, neg_context_words, in_embed, out_embed):
  cw = center_words.astype(jnp.int32)
  pw = pos_context_words.astype(jnp.int32)
  nw = neg_context_words.astype(jnp.int32).reshape(_B * _K)
  rin, rout = _to_rows(in_embed, out_embed)
  scores = _sc_gather(cw, pw, nw, rin, rout)
  return _loss_call(scores.reshape(_NROW * _B // 128, 128)).reshape(())


# speedup vs baseline: 2.2894x; 1.1334x over previous
import jax
import jax.numpy as jnp
from jax import lax
from jax.experimental import pallas as pl
from jax.experimental.pallas import tpu as pltpu
from jax.experimental.pallas import tpu_sc as plsc

_V = 1000000
_B = 16384
_K = 20
_D = 16
_L = 16
_NC = 2
_NS = 16
_NW = _NC * _NS
_BPW = _B // _NW
_CB = 32
_NCHUNK = _BPW // _CB
_NIW = 128
_NJ = _CB * _K // _NIW
_NROW = 1 + _K
_SCHUNK = _NROW * _CB


def _sc_body(cw_hbm, pw_hbm, nw_hbm, in_hbm, out_hbm, sc_hbm,
             cidx_v, pidx_v, nidx_v, cg_v, pg_v, ng_v,
             crow_v, prow_v, nrow_v, scr_v, sem):
  c = lax.axis_index("c")
  s = lax.axis_index("s")
  wid = s * _NC + c
  base = wid * _BPW

  @pl.loop(0, _NCHUNK)
  def _chunk(ci):
    b0 = base + ci * _CB
    pltpu.sync_copy(cw_hbm.at[pl.ds(b0, _CB)], cidx_v)
    pltpu.sync_copy(pw_hbm.at[pl.ds(b0, _CB)], pidx_v)
    pltpu.sync_copy(nw_hbm.at[pl.ds(b0 * _K, _CB * _K)], nidx_v)
    for t in range(_CB // _L):
      cg_v[pl.ds(t * _L, _L)] = cidx_v[pl.ds(t * _L, _L)] >> 3
      pg_v[pl.ds(t * _L, _L)] = pidx_v[pl.ds(t * _L, _L)] >> 3
    for t in range(_CB * _K // _L):
      ng_v[pl.ds(t * _L, _L)] = nidx_v[pl.ds(t * _L, _L)] >> 3
    copies = [
        pltpu.async_copy(in_hbm.at[cg_v], crow_v, sem),
        pltpu.async_copy(out_hbm.at[pg_v], prow_v, sem),
    ]
    for j in range(_NJ):
      copies.append(pltpu.async_copy(
          out_hbm.at[ng_v.at[pl.ds(j * _NIW, _NIW)]],
          nrow_v.at[pl.ds(j * _NIW, _NIW)], sem))
    for cp in copies:
      cp.wait()

    @pl.loop(0, _CB // _L)
    def _group(g):
      row0 = g * _L
      lane = lax.iota(jnp.int32, 16)
      rowi = row0 + lane
      rowk = rowi * _K
      csub = (cidx_v[pl.ds(row0, _L)] & 7) * _D
      psub = (pidx_v[pl.ds(row0, _L)] & 7) * _D
      ccols = [plsc.load_gather(crow_v, [rowi, csub + d]) for d in range(_D)]

      acc = ccols[0] * plsc.load_gather(prow_v, [rowi, psub])
      for d in range(1, _D):
        acc = acc + ccols[d] * plsc.load_gather(prow_v, [rowi, psub + d])
      scr_v[pl.ds(row0, _L)] = acc

      for k in range(_K):
        ri = rowk + k
        nsub = (plsc.load_gather(nidx_v, [ri]) & 7) * _D
        acc = ccols[0] * plsc.load_gather(nrow_v, [ri, nsub])
        for d in range(1, _D):
          acc = acc + ccols[d] * plsc.load_gather(nrow_v, [ri, nsub + d])
        scr_v[pl.ds((1 + k) * _CB + row0, _L)] = -acc

    pltpu.sync_copy(scr_v, sc_hbm.at[pl.ds((wid * _NCHUNK + ci) * _SCHUNK,
                                           _SCHUNK)])


_sc_gather = pl.kernel(
    _sc_body,
    out_type=jax.ShapeDtypeStruct((_NROW * _B,), jnp.float32),
    mesh=plsc.VectorSubcoreMesh(core_axis_name="c", subcore_axis_name="s"),
    compiler_params=pltpu.CompilerParams(
        needs_layout_passes=False, use_tc_tiling_on_sc=True),
    scratch_types=[
        pltpu.VMEM((_CB,), jnp.int32),
        pltpu.VMEM((_CB,), jnp.int32),
        pltpu.VMEM((_CB * _K,), jnp.int32),
        pltpu.VMEM((_CB,), jnp.int32),
        pltpu.VMEM((_CB,), jnp.int32),
        pltpu.VMEM((_CB * _K,), jnp.int32),
        pltpu.VMEM((_CB, 128), jnp.float32),
        pltpu.VMEM((_CB, 128), jnp.float32),
        pltpu.VMEM((_CB * _K, 128), jnp.float32),
        pltpu.VMEM((_SCHUNK,), jnp.float32),
        pltpu.SemaphoreType.DMA,
    ],
)


_NT = _V // 128
_TAIL = _V - _NT * 128


_TPW = _NT // _NW
_NTB = 8


def _cv_group(tabs, tbs, obs, sem, t0, ntb, lane):
  copies = []
  for (wt, _), tb in zip(tabs, tbs):
    for dg in (0, 1):
      copies.append(pltpu.async_copy(
          wt.at[pl.ds(dg * 8, 8), pl.ds(t0 * 128, ntb * 128)],
          tb.at[pl.ds(dg * 8, 8), pl.ds(0, ntb * 128)], sem))
  for cp in copies:
    cp.wait()

  @pl.loop(0, ntb)
  def _tt(tt):
    for tb, ob in zip(tbs, obs):
      for v0 in range(0, 128, _L):
        ridx = tt * _D + ((v0 + lane) >> 3)
        cidx = ((v0 + lane) & 7) * _D
        for d in range(_D):
          plsc.store_scatter(ob, [ridx, cidx + d],
                             tb[d, pl.ds(tt * 128 + v0, _L)])

  for (_, r), ob in zip(tabs, obs):
    pltpu.sync_copy(ob.at[pl.ds(0, ntb * _D)], r.at[pl.ds(t0 * _D, ntb * _D)])


def _cv_body(wti_hbm, wto_hbm, tli_hbm, tlo_hbm, ri_hbm, ro_hbm,
             tbi0_v, tbo0_v, tbi1_v, tbo1_v, obi_v, obo_v, tl_v, sem):
  c = lax.axis_index("c")
  s = lax.axis_index("s")
  wid = s * _NC + c
  base = wid * _TPW
  lane = lax.iota(jnp.int32, 16)
  tabs = ((wti_hbm, ri_hbm), (wto_hbm, ro_hbm))
  bufs = ((tbi0_v, tbo0_v), (tbi1_v, tbo1_v))
  obs = (obi_v, obo_v)
  ng = _TPW // _NTB

  def _fire(t0, tbs):
    for (wt, _), tb in zip(tabs, tbs):
      for dg in (0, 1):
        pltpu.async_copy(
            wt.at[pl.ds(dg * 8, 8), pl.ds(t0 * 128, _NTB * 128)],
            tb.at[pl.ds(dg * 8, 8)], sem)

  def _drain(tbs):
    for (wt, _), tb in zip(tabs, tbs):
      for dg in (0, 1):
        pltpu.make_async_copy(
            wt.at[pl.ds(dg * 8, 8), pl.ds(0, _NTB * 128)],
            tb.at[pl.ds(dg * 8, 8)], sem).wait()

  def _compute(t0, tbs):
    @pl.loop(0, _NTB)
    def _tt(tt):
      for tb, ob in zip(tbs, obs):
        for v0 in range(0, 128, _L):
          ridx = tt * _D + ((v0 + lane) >> 3)
          cidx = ((v0 + lane) & 7) * _D
          for d in range(_D):
            plsc.store_scatter(ob, [ridx, cidx + d],
                               tb[d, pl.ds(tt * 128 + v0, _L)])
    for (_, r), ob in zip(tabs, obs):
      pltpu.sync_copy(ob, r.at[pl.ds(t0 * _D, _NTB * _D)])

  _fire(base, bufs[0])

  @pl.loop(0, ng, step=2)
  def _g(g):
    for b in (0, 1):
      gi = g + b

      @pl.when(gi + 1 < ng)
      def _prefetch():
        _fire(base + (gi + 1) * _NTB, bufs[1 - b])

      _drain(bufs[b])
      _compute(base + gi * _NTB, bufs[b])

  rem = _TPW % _NTB
  if rem:
    _cv_group(tabs, bufs[0], obs, sem, base + _TPW - rem, rem, lane)

  @pl.when(wid == _NW - 1)
  def _tail_blk():
    _cv_group(tabs, bufs[0], obs, sem, _NW * _TPW, _NT - _NW * _TPW, lane)
    for (tl, r), ob in zip(((tli_hbm, ri_hbm), (tlo_hbm, ro_hbm)), obs):
      pltpu.sync_copy(tl, tl_v)
      for lv in range(_TAIL):
        plsc.store_scatter(
            ob,
            [jnp.full((16,), lv >> 3, dtype=jnp.int32),
             (lv & 7) * _D + lane],
            tl_v[lv, :])
      pltpu.sync_copy(ob.at[pl.ds(0, _TAIL // 8)],
                      r.at[pl.ds(_NT * _D, _TAIL // 8)])


_convert = pl.kernel(
    _cv_body,
    out_type=(jax.ShapeDtypeStruct((_V // 8, 128), jnp.float32),
              jax.ShapeDtypeStruct((_V // 8, 128), jnp.float32)),
    mesh=plsc.VectorSubcoreMesh(core_axis_name="c", subcore_axis_name="s"),
    compiler_params=pltpu.CompilerParams(
        needs_layout_passes=False, use_tc_tiling_on_sc=True),
    scratch_types=[
        pltpu.VMEM((_D, _NTB * 128), jnp.float32),
        pltpu.VMEM((_D, _NTB * 128), jnp.float32),
        pltpu.VMEM((_D, _NTB * 128), jnp.float32),
        pltpu.VMEM((_D, _NTB * 128), jnp.float32),
        pltpu.VMEM((_NTB * _D, 128), jnp.float32),
        pltpu.VMEM((_NTB * _D, 128), jnp.float32),
        pltpu.VMEM((_TAIL, _D), jnp.float32),
        pltpu.SemaphoreType.DMA,
    ],
)


def _to_rows(in_embed, out_embed):
  return _convert(in_embed.T, out_embed.T,
                  in_embed[_NT * 128:], out_embed[_NT * 128:])


def _loss_body(s_ref, o_ref):
  x = s_ref[...]
  o_ref[0, 0] = -jnp.sum(jax.nn.log_sigmoid(x)) / _B


_loss_call = pl.pallas_call(
    _loss_body,
    out_shape=jax.ShapeDtypeStruct((1, 1), jnp.float32),
    out_specs=pl.BlockSpec(memory_space=pltpu.SMEM),
)


def kernel(center_words, pos_context_words, neg_context_words, in_embed, out_embed):
  cw = center_words.astype(jnp.int32)
  pw = pos_context_words.astype(jnp.int32)
  nw = neg_context_words.astype(jnp.int32).reshape(_B * _K)
  rin, rout = _to_rows(in_embed, out_embed)
  scores = _sc_gather(cw, pw, nw, rin, rout)
  return _loss_call(scores.reshape(_NROW * _B // 128, 128)).reshape(())


# speedup vs baseline: 2.5088x; 1.0958x over previous
import jax
import jax.numpy as jnp
from jax import lax
from jax.experimental import pallas as pl
from jax.experimental.pallas import tpu as pltpu
from jax.experimental.pallas import tpu_sc as plsc

_V = 1000000
_B = 16384
_K = 20
_D = 16
_L = 16
_NC = 2
_NS = 16
_NW = _NC * _NS
_BPW = _B // _NW
_CB = 32
_NCHUNK = _BPW // _CB
_NIW = 128
_NJ = _CB * _K // _NIW
_NROW = 1 + _K
_SCHUNK = _NROW * _CB


def _sc_body(cw_hbm, pw_hbm, nw_hbm, in_hbm, out_hbm, sc_hbm,
             cidx_v, pidx_v, nidx_v, cg_v, pg_v, ng_v,
             crow_v, prow_v, nrow_v, scr_v, sem):
  c = lax.axis_index("c")
  s = lax.axis_index("s")
  wid = s * _NC + c
  base = wid * _BPW

  @pl.loop(0, _NCHUNK)
  def _chunk(ci):
    b0 = base + ci * _CB
    pltpu.sync_copy(cw_hbm.at[pl.ds(b0, _CB)], cidx_v)
    pltpu.sync_copy(pw_hbm.at[pl.ds(b0, _CB)], pidx_v)
    pltpu.sync_copy(nw_hbm.at[pl.ds(b0 * _K, _CB * _K)], nidx_v)
    for t in range(_CB // _L):
      cg_v[pl.ds(t * _L, _L)] = cidx_v[pl.ds(t * _L, _L)] >> 3
      pg_v[pl.ds(t * _L, _L)] = pidx_v[pl.ds(t * _L, _L)] >> 3
    for t in range(_CB * _K // _L):
      ng_v[pl.ds(t * _L, _L)] = nidx_v[pl.ds(t * _L, _L)] >> 3
    copies = [
        pltpu.async_copy(in_hbm.at[cg_v], crow_v, sem),
        pltpu.async_copy(out_hbm.at[pg_v], prow_v, sem),
    ]
    for j in range(_NJ):
      copies.append(pltpu.async_copy(
          out_hbm.at[ng_v.at[pl.ds(j * _NIW, _NIW)]],
          nrow_v.at[pl.ds(j * _NIW, _NIW)], sem))
    for cp in copies:
      cp.wait()

    @pl.loop(0, _CB // _L)
    def _group(g):
      row0 = g * _L
      lane = lax.iota(jnp.int32, 16)
      rowi = row0 + lane
      rowk = rowi * _K
      csub = (cidx_v[pl.ds(row0, _L)] & 7) * _D
      psub = (pidx_v[pl.ds(row0, _L)] & 7) * _D
      ccols = [plsc.load_gather(crow_v, [rowi, csub + d]) for d in range(_D)]

      acc = ccols[0] * plsc.load_gather(prow_v, [rowi, psub])
      for d in range(1, _D):
        acc = acc + ccols[d] * plsc.load_gather(prow_v, [rowi, psub + d])
      scr_v[pl.ds(row0, _L)] = acc

      for k in range(_K):
        ri = rowk + k
        nsub = (plsc.load_gather(nidx_v, [ri]) & 7) * _D
        acc = ccols[0] * plsc.load_gather(nrow_v, [ri, nsub])
        for d in range(1, _D):
          acc = acc + ccols[d] * plsc.load_gather(nrow_v, [ri, nsub + d])
        scr_v[pl.ds((1 + k) * _CB + row0, _L)] = -acc

    pltpu.sync_copy(scr_v, sc_hbm.at[pl.ds((wid * _NCHUNK + ci) * _SCHUNK,
                                           _SCHUNK)])


_sc_gather = pl.kernel(
    _sc_body,
    out_type=jax.ShapeDtypeStruct((_NROW * _B,), jnp.float32),
    mesh=plsc.VectorSubcoreMesh(core_axis_name="c", subcore_axis_name="s"),
    compiler_params=pltpu.CompilerParams(
        needs_layout_passes=False, use_tc_tiling_on_sc=True),
    scratch_types=[
        pltpu.VMEM((_CB,), jnp.int32),
        pltpu.VMEM((_CB,), jnp.int32),
        pltpu.VMEM((_CB * _K,), jnp.int32),
        pltpu.VMEM((_CB,), jnp.int32),
        pltpu.VMEM((_CB,), jnp.int32),
        pltpu.VMEM((_CB * _K,), jnp.int32),
        pltpu.VMEM((_CB, 128), jnp.float32),
        pltpu.VMEM((_CB, 128), jnp.float32),
        pltpu.VMEM((_CB * _K, 128), jnp.float32),
        pltpu.VMEM((_SCHUNK,), jnp.float32),
        pltpu.SemaphoreType.DMA,
    ],
)


_NT = _V // 128
_TAIL = _V - _NT * 128


_TPW = _NT // _NW
_NTB = 6


def _cv_group(tabs, tbs, obs, sem, t0, ntb, lane):
  copies = []
  for (wt, _), tb in zip(tabs, tbs):
    for dg in (0, 1):
      copies.append(pltpu.async_copy(
          wt.at[pl.ds(dg * 8, 8), pl.ds(t0 * 128, ntb * 128)],
          tb.at[pl.ds(dg * 8, 8), pl.ds(0, ntb * 128)], sem))
  for cp in copies:
    cp.wait()

  @pl.loop(0, ntb)
  def _tt(tt):
    for tb, ob in zip(tbs, obs):
      for v0 in range(0, 128, _L):
        ridx = tt * _D + ((v0 + lane) >> 3)
        cidx = ((v0 + lane) & 7) * _D
        for d in range(_D):
          plsc.store_scatter(ob, [ridx, cidx + d],
                             tb[d, pl.ds(tt * 128 + v0, _L)])

  for (_, r), ob in zip(tabs, obs):
    pltpu.sync_copy(ob.at[pl.ds(0, ntb * _D)], r.at[pl.ds(t0 * _D, ntb * _D)])


def _cv_body(wti_hbm, wto_hbm, tli_hbm, tlo_hbm, ri_hbm, ro_hbm,
             tbi0_v, tbo0_v, tbi1_v, tbo1_v,
             obi0_v, obo0_v, obi1_v, obo1_v, tl_v, sem, osem):
  c = lax.axis_index("c")
  s = lax.axis_index("s")
  wid = s * _NC + c
  base = wid * _TPW
  lane = lax.iota(jnp.int32, 16)
  tabs = ((wti_hbm, ri_hbm), (wto_hbm, ro_hbm))
  bufs = ((tbi0_v, tbo0_v), (tbi1_v, tbo1_v))
  obufs = ((obi0_v, obo0_v), (obi1_v, obo1_v))
  ng = _TPW // _NTB

  def _fire(t0, tbs):
    for (wt, _), tb in zip(tabs, tbs):
      for dg in (0, 1):
        pltpu.async_copy(
            wt.at[pl.ds(dg * 8, 8), pl.ds(t0 * 128, _NTB * 128)],
            tb.at[pl.ds(dg * 8, 8)], sem)

  def _drain(tbs):
    for (wt, _), tb in zip(tabs, tbs):
      for dg in (0, 1):
        pltpu.make_async_copy(
            wt.at[pl.ds(dg * 8, 8), pl.ds(0, _NTB * 128)],
            tb.at[pl.ds(dg * 8, 8)], sem).wait()

  def _drain_out(obs):
    for (_, r), ob in zip(tabs, obs):
      pltpu.make_async_copy(ob, r.at[pl.ds(0, _NTB * _D)], osem).wait()

  def _compute(t0, tbs, obs):
    @pl.loop(0, _NTB)
    def _tt(tt):
      for tb, ob in zip(tbs, obs):
        for v0 in range(0, 128, _L):
          ridx = tt * _D + ((v0 + lane) >> 3)
          cidx = ((v0 + lane) & 7) * _D
          for d in range(_D):
            plsc.store_scatter(ob, [ridx, cidx + d],
                               tb[d, pl.ds(tt * 128 + v0, _L)])
    for (_, r), ob in zip(tabs, obs):
      pltpu.async_copy(ob, r.at[pl.ds(t0 * _D, _NTB * _D)], osem)

  _fire(base, bufs[0])

  @pl.loop(0, ng, step=2)
  def _g(g):
    for b in (0, 1):
      gi = g + b

      @pl.when(gi + 1 < ng)
      def _prefetch():
        _fire(base + (gi + 1) * _NTB, bufs[1 - b])

      _drain(bufs[b])

      @pl.when(gi >= 2)
      def _settle():
        _drain_out(obufs[b])

      _compute(base + gi * _NTB, bufs[b], obufs[b])

  _drain_out(obufs[0])
  _drain_out(obufs[1])

  rem = _TPW % _NTB
  if rem:
    _cv_group(tabs, bufs[0], obufs[0], sem, base + _TPW - rem, rem, lane)

  @pl.when(wid == _NW - 1)
  def _tail_blk():
    _cv_group(tabs, bufs[0], obufs[0], sem, _NW * _TPW, _NT - _NW * _TPW, lane)
    for (tl, r), ob in zip(((tli_hbm, ri_hbm), (tlo_hbm, ro_hbm)), obufs[0]):
      pltpu.sync_copy(tl, tl_v)
      for lv in range(_TAIL):
        plsc.store_scatter(
            ob,
            [jnp.full((16,), lv >> 3, dtype=jnp.int32),
             (lv & 7) * _D + lane],
            tl_v[lv, :])
      pltpu.sync_copy(ob.at[pl.ds(0, _TAIL // 8)],
                      r.at[pl.ds(_NT * _D, _TAIL // 8)])


_convert = pl.kernel(
    _cv_body,
    out_type=(jax.ShapeDtypeStruct((_V // 8, 128), jnp.float32),
              jax.ShapeDtypeStruct((_V // 8, 128), jnp.float32)),
    mesh=plsc.VectorSubcoreMesh(core_axis_name="c", subcore_axis_name="s"),
    compiler_params=pltpu.CompilerParams(
        needs_layout_passes=False, use_tc_tiling_on_sc=True),
    scratch_types=[
        pltpu.VMEM((_D, _NTB * 128), jnp.float32),
        pltpu.VMEM((_D, _NTB * 128), jnp.float32),
        pltpu.VMEM((_D, _NTB * 128), jnp.float32),
        pltpu.VMEM((_D, _NTB * 128), jnp.float32),
        pltpu.VMEM((_NTB * _D, 128), jnp.float32),
        pltpu.VMEM((_NTB * _D, 128), jnp.float32),
        pltpu.VMEM((_NTB * _D, 128), jnp.float32),
        pltpu.VMEM((_NTB * _D, 128), jnp.float32),
        pltpu.VMEM((_TAIL, _D), jnp.float32),
        pltpu.SemaphoreType.DMA,
        pltpu.SemaphoreType.DMA,
    ],
)


def _to_rows(in_embed, out_embed):
  return _convert(in_embed.T, out_embed.T,
                  in_embed[_NT * 128:], out_embed[_NT * 128:])


def _loss_body(s_ref, o_ref):
  x = s_ref[...]
  o_ref[0, 0] = -jnp.sum(jax.nn.log_sigmoid(x)) / _B


_loss_call = pl.pallas_call(
    _loss_body,
    out_shape=jax.ShapeDtypeStruct((1, 1), jnp.float32),
    out_specs=pl.BlockSpec(memory_space=pltpu.SMEM),
)


def kernel(center_words, pos_context_words, neg_context_words, in_embed, out_embed):
  cw = center_words.astype(jnp.int32)
  pw = pos_context_words.astype(jnp.int32)
  nw = neg_context_words.astype(jnp.int32).reshape(_B * _K)
  rin, rout = _to_rows(in_embed, out_embed)
  scores = _sc_gather(cw, pw, nw, rin, rout)
  return _loss_call(scores.reshape(_NROW * _B // 128, 128)).reshape(())


# speedup vs baseline: 2.8028x; 1.1172x over previous
import jax
import jax.numpy as jnp
from jax import lax
from jax.experimental import pallas as pl
from jax.experimental.pallas import tpu as pltpu
from jax.experimental.pallas import tpu_sc as plsc

_V = 1000000
_B = 16384
_K = 20
_D = 16
_L = 16
_NC = 2
_NS = 16
_NW = _NC * _NS
_BPW = _B // _NW
_CB = 16
_NCHUNK = _BPW // _CB
_NIW = 128
_NJ = _CB * _K // _NIW
_NROW = 1 + _K
_SCHUNK = _NROW * _CB


def _sc_body(cw_hbm, pw_hbm, nw_hbm, in_hbm, out_hbm, sc_hbm,
             cidx_v, pidx_v, nidx_v, cg_v, pg_v, ng_v,
             crow0_v, prow0_v, nrow0_v, crow1_v, prow1_v, nrow1_v,
             scr0_v, scr1_v, sem, osem):
  c = lax.axis_index("c")
  s = lax.axis_index("s")
  wid = s * _NC + c
  base = wid * _BPW
  lane = lax.iota(jnp.int32, 16)
  rows = ((crow0_v, prow0_v, nrow0_v), (crow1_v, prow1_v, nrow1_v))
  scrs = (scr0_v, scr1_v)

  pltpu.sync_copy(cw_hbm.at[pl.ds(base, _BPW)], cidx_v)
  pltpu.sync_copy(pw_hbm.at[pl.ds(base, _BPW)], pidx_v)
  pltpu.sync_copy(nw_hbm.at[pl.ds(base * _K, _BPW * _K)], nidx_v)
  for t in range(_BPW // _L):
    cg_v[pl.ds(t * _L, _L)] = cidx_v[pl.ds(t * _L, _L)] >> 3
    pg_v[pl.ds(t * _L, _L)] = pidx_v[pl.ds(t * _L, _L)] >> 3

  @pl.loop(0, _BPW * _K // _L)
  def _sh(t):
    ng_v[pl.ds(t * _L, _L)] = nidx_v[pl.ds(t * _L, _L)] >> 3

  nsplit = []
  off = 0
  while off < _CB * _K:
    w = min(_NIW, _CB * _K - off)
    nsplit.append((off, w))
    off += w

  def _fire(ci, bufs):
    cr, pr, nr = bufs
    pltpu.async_copy(in_hbm.at[cg_v.at[pl.ds(ci * _CB, _CB)]], cr, sem)
    pltpu.async_copy(out_hbm.at[pg_v.at[pl.ds(ci * _CB, _CB)]], pr, sem)
    for (o, w) in nsplit:
      pltpu.async_copy(out_hbm.at[ng_v.at[pl.ds(ci * _CB * _K + o, w)]],
                       nr.at[pl.ds(o, w)], sem)

  def _drain(bufs):
    cr, pr, nr = bufs
    pltpu.make_async_copy(in_hbm.at[cg_v.at[pl.ds(0, _CB)]], cr, sem).wait()
    pltpu.make_async_copy(in_hbm.at[cg_v.at[pl.ds(0, _CB)]], pr, sem).wait()
    for (o, w) in nsplit:
      pltpu.make_async_copy(out_hbm.at[ng_v.at[pl.ds(o, w)]],
                            nr.at[pl.ds(o, w)], sem).wait()

  def _compute(ci, bufs, scr):
    cr, pr, nr = bufs
    for g in range(_CB // _L):
      row0 = g * _L
      rowi = row0 + lane
      rowk = rowi * _K
      i0 = ci * _CB + row0
      csub = (cidx_v[pl.ds(i0, _L)] & 7) * _D
      psub = (pidx_v[pl.ds(i0, _L)] & 7) * _D
      ccols = [plsc.load_gather(cr, [rowi, csub + d]) for d in range(_D)]

      acc = ccols[0] * plsc.load_gather(pr, [rowi, psub])
      for d in range(1, _D):
        acc = acc + ccols[d] * plsc.load_gather(pr, [rowi, psub + d])
      scr[pl.ds(row0, _L)] = acc

      for k in range(_K):
        ri = rowk + k
        nsub = (plsc.load_gather(nidx_v, [ci * _CB * _K + ri]) & 7) * _D
        acc = ccols[0] * plsc.load_gather(nr, [ri, nsub])
        for d in range(1, _D):
          acc = acc + ccols[d] * plsc.load_gather(nr, [ri, nsub + d])
        scr[pl.ds((1 + k) * _CB + row0, _L)] = -acc
    pltpu.async_copy(
        scr, sc_hbm.at[pl.ds((wid * _NCHUNK + ci) * _SCHUNK, _SCHUNK)], osem)

  def _drain_scr(scr):
    pltpu.make_async_copy(scr, sc_hbm.at[pl.ds(0, _SCHUNK)], osem).wait()

  _fire(0, rows[0])

  @pl.loop(0, _NCHUNK, step=2)
  def _chunk(cc):
    for b in (0, 1):
      ci = cc + b

      @pl.when(ci + 1 < _NCHUNK)
      def _prefetch():
        _fire(ci + 1, rows[1 - b])

      _drain(rows[b])

      @pl.when(ci >= 2)
      def _settle():
        _drain_scr(scrs[b])

      _compute(ci, rows[b], scrs[b])

  _drain_scr(scrs[0])
  _drain_scr(scrs[1])


_sc_gather = pl.kernel(
    _sc_body,
    out_type=jax.ShapeDtypeStruct((_NROW * _B,), jnp.float32),
    mesh=plsc.VectorSubcoreMesh(core_axis_name="c", subcore_axis_name="s"),
    compiler_params=pltpu.CompilerParams(
        needs_layout_passes=False, use_tc_tiling_on_sc=True),
    scratch_types=[
        pltpu.VMEM((_BPW,), jnp.int32),
        pltpu.VMEM((_BPW,), jnp.int32),
        pltpu.VMEM((_BPW * _K,), jnp.int32),
        pltpu.VMEM((_BPW,), jnp.int32),
        pltpu.VMEM((_BPW,), jnp.int32),
        pltpu.VMEM((_BPW * _K,), jnp.int32),
        pltpu.VMEM((_CB, 128), jnp.float32),
        pltpu.VMEM((_CB, 128), jnp.float32),
        pltpu.VMEM((_CB * _K, 128), jnp.float32),
        pltpu.VMEM((_CB, 128), jnp.float32),
        pltpu.VMEM((_CB, 128), jnp.float32),
        pltpu.VMEM((_CB * _K, 128), jnp.float32),
        pltpu.VMEM((_SCHUNK,), jnp.float32),
        pltpu.VMEM((_SCHUNK,), jnp.float32),
        pltpu.SemaphoreType.DMA,
        pltpu.SemaphoreType.DMA,
    ],
)


_NT = _V // 128
_TAIL = _V - _NT * 128


_TPW = _NT // _NW
_NTB = 6


def _cv_group(tabs, tbs, obs, sem, t0, ntb, lane):
  copies = []
  for (wt, _), tb in zip(tabs, tbs):
    for dg in (0, 1):
      copies.append(pltpu.async_copy(
          wt.at[pl.ds(dg * 8, 8), pl.ds(t0 * 128, ntb * 128)],
          tb.at[pl.ds(dg * 8, 8), pl.ds(0, ntb * 128)], sem))
  for cp in copies:
    cp.wait()

  @pl.loop(0, ntb)
  def _tt(tt):
    for tb, ob in zip(tbs, obs):
      for v0 in range(0, 128, _L):
        ridx = tt * _D + ((v0 + lane) >> 3)
        cidx = ((v0 + lane) & 7) * _D
        for d in range(_D):
          plsc.store_scatter(ob, [ridx, cidx + d],
                             tb[d, pl.ds(tt * 128 + v0, _L)])

  for (_, r), ob in zip(tabs, obs):
    pltpu.sync_copy(ob.at[pl.ds(0, ntb * _D)], r.at[pl.ds(t0 * _D, ntb * _D)])


def _cv_body(wti_hbm, wto_hbm, tli_hbm, tlo_hbm, ri_hbm, ro_hbm,
             tbi0_v, tbo0_v, tbi1_v, tbo1_v,
             obi0_v, obo0_v, obi1_v, obo1_v, tl_v, sem, osem):
  c = lax.axis_index("c")
  s = lax.axis_index("s")
  wid = s * _NC + c
  base = wid * _TPW
  lane = lax.iota(jnp.int32, 16)
  tabs = ((wti_hbm, ri_hbm), (wto_hbm, ro_hbm))
  bufs = ((tbi0_v, tbo0_v), (tbi1_v, tbo1_v))
  obufs = ((obi0_v, obo0_v), (obi1_v, obo1_v))
  ng = _TPW // _NTB

  def _fire(t0, tbs):
    for (wt, _), tb in zip(tabs, tbs):
      for dg in (0, 1):
        pltpu.async_copy(
            wt.at[pl.ds(dg * 8, 8), pl.ds(t0 * 128, _NTB * 128)],
            tb.at[pl.ds(dg * 8, 8)], sem)

  def _drain(tbs):
    for (wt, _), tb in zip(tabs, tbs):
      for dg in (0, 1):
        pltpu.make_async_copy(
            wt.at[pl.ds(dg * 8, 8), pl.ds(0, _NTB * 128)],
            tb.at[pl.ds(dg * 8, 8)], sem).wait()

  def _drain_out(obs):
    for (_, r), ob in zip(tabs, obs):
      pltpu.make_async_copy(ob, r.at[pl.ds(0, _NTB * _D)], osem).wait()

  def _compute(t0, tbs, obs):
    @pl.loop(0, _NTB)
    def _tt(tt):
      for tb, ob in zip(tbs, obs):
        for v0 in range(0, 128, _L):
          ridx = tt * _D + ((v0 + lane) >> 3)
          cidx = ((v0 + lane) & 7) * _D
          for d in range(_D):
            plsc.store_scatter(ob, [ridx, cidx + d],
                               tb[d, pl.ds(tt * 128 + v0, _L)])
    for (_, r), ob in zip(tabs, obs):
      pltpu.async_copy(ob, r.at[pl.ds(t0 * _D, _NTB * _D)], osem)

  _fire(base, bufs[0])

  @pl.loop(0, ng, step=2)
  def _g(g):
    for b in (0, 1):
      gi = g + b

      @pl.when(gi + 1 < ng)
      def _prefetch():
        _fire(base + (gi + 1) * _NTB, bufs[1 - b])

      _drain(bufs[b])

      @pl.when(gi >= 2)
      def _settle():
        _drain_out(obufs[b])

      _compute(base + gi * _NTB, bufs[b], obufs[b])

  _drain_out(obufs[0])
  _drain_out(obufs[1])

  rem = _TPW % _NTB
  if rem:
    _cv_group(tabs, bufs[0], obufs[0], sem, base + _TPW - rem, rem, lane)

  @pl.when(wid == _NW - 1)
  def _tail_blk():
    _cv_group(tabs, bufs[0], obufs[0], sem, _NW * _TPW, _NT - _NW * _TPW, lane)
    for (tl, r), ob in zip(((tli_hbm, ri_hbm), (tlo_hbm, ro_hbm)), obufs[0]):
      pltpu.sync_copy(tl, tl_v)
      for lv in range(_TAIL):
        plsc.store_scatter(
            ob,
            [jnp.full((16,), lv >> 3, dtype=jnp.int32),
             (lv & 7) * _D + lane],
            tl_v[lv, :])
      pltpu.sync_copy(ob.at[pl.ds(0, _TAIL // 8)],
                      r.at[pl.ds(_NT * _D, _TAIL // 8)])


_convert = pl.kernel(
    _cv_body,
    out_type=(jax.ShapeDtypeStruct((_V // 8, 128), jnp.float32),
              jax.ShapeDtypeStruct((_V // 8, 128), jnp.float32)),
    mesh=plsc.VectorSubcoreMesh(core_axis_name="c", subcore_axis_name="s"),
    compiler_params=pltpu.CompilerParams(
        needs_layout_passes=False, use_tc_tiling_on_sc=True),
    scratch_types=[
        pltpu.VMEM((_D, _NTB * 128), jnp.float32),
        pltpu.VMEM((_D, _NTB * 128), jnp.float32),
        pltpu.VMEM((_D, _NTB * 128), jnp.float32),
        pltpu.VMEM((_D, _NTB * 128), jnp.float32),
        pltpu.VMEM((_NTB * _D, 128), jnp.float32),
        pltpu.VMEM((_NTB * _D, 128), jnp.float32),
        pltpu.VMEM((_NTB * _D, 128), jnp.float32),
        pltpu.VMEM((_NTB * _D, 128), jnp.float32),
        pltpu.VMEM((_TAIL, _D), jnp.float32),
        pltpu.SemaphoreType.DMA,
        pltpu.SemaphoreType.DMA,
    ],
)


def _to_rows(in_embed, out_embed):
  return _convert(in_embed.T, out_embed.T,
                  in_embed[_NT * 128:], out_embed[_NT * 128:])


def _loss_body(s_ref, o_ref):
  x = s_ref[...]
  o_ref[0, 0] = -jnp.sum(jax.nn.log_sigmoid(x)) / _B


_loss_call = pl.pallas_call(
    _loss_body,
    out_shape=jax.ShapeDtypeStruct((1, 1), jnp.float32),
    out_specs=pl.BlockSpec(memory_space=pltpu.SMEM),
)


def kernel(center_words, pos_context_words, neg_context_words, in_embed, out_embed):
  cw = center_words.astype(jnp.int32)
  pw = pos_context_words.astype(jnp.int32)
  nw = neg_context_words.astype(jnp.int32).reshape(_B * _K)
  rin, rout = _to_rows(in_embed, out_embed)
  scores = _sc_gather(cw, pw, nw, rin, rout)
  return _loss_call(scores.reshape(_NROW * _B // 128, 128)).reshape(())
